# baseline (speedup 1.0000x reference)
import functools
import math

import jax
import jax.numpy as jnp
from jax import lax
from jax.experimental import pallas as pl
from jax.experimental.pallas import tpu as pltpu
from jax.experimental.pallas import tpu_sc as plsc

N = 4096
D_FEAT = 128
D_HID = 64
T_TOK = 77
D_TXT = 768
N_SEG = 32
SEG_LEN = 64

ROW_BLK = 512
N_BLK = N // ROW_BLK

_F32 = jnp.float32


def _bf(x):
    return x.astype(jnp.bfloat16)


def _mm(a, b):
    return jax.lax.dot_general(_bf(a), _bf(b), (((1,), (0,)), ((), ())),
                               preferred_element_type=_F32)


def _mega_body(frames_ref, text_ref, w1_ref, w2_ref,
               wih_ref, whh_ref, bih_ref, bhh_ref,
               wmf_ref, wt0_ref, we2t0_ref, wmt0_ref, we2f0_ref, wf0_ref,
               wmt1_ref, we2f1_ref, wf1_ref,
               f1_out_ref,
               bi_s, y_s, wf_s, sqc_s, sqr_s, gi_s):
    xw = _mm(frames_ref[...], w1_ref[...])
    row = lax.broadcasted_iota(jnp.int32, (N, 1), 0)
    edge = (row == 0) | (row == N - 1)
    dinv = 1.0 / jnp.sqrt(jnp.where(edge, jnp.float32(2.0), jnp.float32(3.0)))
    zero = jnp.zeros((1, 1), _F32)
    dinv_up = jnp.concatenate([zero, dinv[:-1, :]], axis=0)
    dinv_dn = jnp.concatenate([dinv[1:, :], zero], axis=0)
    c_up = _bf(dinv * dinv_up).astype(_F32)
    c_self = _bf(dinv * dinv).astype(_F32)
    c_dn = _bf(dinv * dinv_dn).astype(_F32)
    xb = _bf(xw).astype(_F32)
    zrow = jnp.zeros((1, D_HID), _F32)
    x_up = jnp.concatenate([zrow, xb[:-1, :]], axis=0)
    x_dn = jnp.concatenate([xb[1:, :], zrow], axis=0)
    bi = jax.nn.relu(c_up * x_up + c_self * xb + c_dn * x_dn)
    bi_s[...] = bi
    y_s[...] = _mm(bi, w2_ref[...])
    sq = jnp.sum(bi * bi, axis=1)
    sqc_s[...] = sq[:, None]
    sqr_s[...] = sq[None, :]

    def dist_blk(i):
        off = pl.multiple_of(i * ROW_BLK, ROW_BLK)
        xb = bi_s[pl.ds(off, ROW_BLK), :]
        sqb = sqc_s[pl.ds(off, ROW_BLK), :]
        xxt = jax.lax.dot_general(xb, bi_s[...], (((1,), (1,)), ((), ())),
                                  preferred_element_type=_F32)
        d2 = sqb + sqr_s[...] - 2.0 * xxt
        return jnp.sqrt(jnp.maximum(d2, 0.0))

    def pass_a(i, dsum):
        return dsum + jnp.sum(dist_blk(i))

    dsum = lax.fori_loop(0, N_BLK, pass_a, jnp.float32(0.0))
    stat = dsum * (1.0 / (N * N))

    def pass_b(i, m0):
        off = pl.multiple_of(i * ROW_BLK, ROW_BLK)
        adj = jnp.exp(-dist_blk(i) / (stat + 1e-6))
        rowsum = jnp.sum(adj, axis=1, keepdims=True)
        adjn = adj / (rowsum + 1e-6)
        wfb = jax.nn.relu(_mm(adjn, y_s[...]))
        wf_s[pl.ds(off, ROW_BLK), :] = wfb
        return m0 + jnp.sum(jax.nn.relu(_mm(wfb, wmf_ref[...])),
                            axis=0, keepdims=True)

    m0 = lax.fori_loop(0, N_BLK, pass_b, jnp.zeros((1, D_HID), _F32))

    gi_s[...] = _mm(text_ref[...], wih_ref[...]) + bih_ref[...]
    whh = whh_ref[...]
    bhh = bhh_ref[...]
    H = D_HID

    def gru_step(i, h):
        gi = gi_s[pl.ds(i, 1), :]
        gh = _mm(h, whh) + bhh
        r = jax.nn.sigmoid(gi[:, :H] + gh[:, :H])
        z = jax.nn.sigmoid(gi[:, H:2 * H] + gh[:, H:2 * H])
        n = jnp.tanh(gi[:, 2 * H:] + r * gh[:, 2 * H:])
        return (1.0 - z) * n + z * h

    t = lax.fori_loop(0, T_TOK, gru_step, jnp.zeros((1, H), _F32))

    m0 = m0 * (1.0 / N)
    t0 = jax.nn.relu(_mm(t, wt0_ref[...]) + _mm(m0, we2t0_ref[...]))
    bias0 = _mm(jax.nn.relu(_mm(t, wmt0_ref[...])), we2f0_ref[...])
    f0 = jax.nn.relu(_mm(wf_s[...], wf0_ref[...]) + bias0)
    bias1 = _mm(jax.nn.relu(_mm(t0, wmt1_ref[...])), we2f1_ref[...])
    f1_out_ref[...] = jax.nn.relu(_mm(f0, wf1_ref[...]) + bias1)


def _sc_round_bf16(v):
    u = lax.bitcast_convert_type(v, jnp.uint32)
    lsb = (u >> jnp.uint32(16)) & jnp.uint32(1)
    r = (u + jnp.uint32(0x7FFF) + lsb) & jnp.uint32(0xFFFF0000)
    return lax.bitcast_convert_type(r, _F32)


def _seg_pool_sc(seg_idx, f1, w_fc_vec, b_fc16):
    mesh = plsc.VectorSubcoreMesh(core_axis_name="c", subcore_axis_name="s")

    @functools.partial(
        pl.kernel, mesh=mesh,
        compiler_params=pltpu.CompilerParams(needs_layout_passes=False,
                                             use_tc_tiling_on_sc=False),
        out_type=jax.ShapeDtypeStruct((N_SEG, 16), _F32),
        scratch_types=[
            pltpu.VMEM((SEG_LEN,), jnp.int32),
            pltpu.VMEM((SEG_LEN, D_HID), _F32),
            pltpu.VMEM((D_HID,), _F32),
            pltpu.VMEM((16,), _F32),
            pltpu.VMEM((16,), _F32),
            pltpu.SemaphoreType.DMA,
        ],
    )
    def seg_kernel(idx_hbm, f_hbm, wfc_hbm, bfc_hbm, out_hbm,
                   idx_v, rows_v, wfc_v, bfc_v, res_v, sem):
        wid = lax.axis_index("s") * 2 + lax.axis_index("c")
        pltpu.sync_copy(wfc_hbm, wfc_v)
        pltpu.sync_copy(bfc_hbm, bfc_v)
        pltpu.sync_copy(idx_hbm.at[wid], idx_v)
        pltpu.async_copy(f_hbm.at[idx_v], rows_v, sem).wait()
        prod = jnp.zeros((16,), _F32)
        for c in range(D_HID // 16):
            cs = jnp.zeros((16,), _F32)
            for r in range(SEG_LEN):
                cs = cs + rows_v[r, pl.ds(c * 16, 16)]
            segb = _sc_round_bf16(cs * (1.0 / SEG_LEN))
            prod = prod + segb * _sc_round_bf16(wfc_v[pl.ds(c * 16, 16)])
        score = jnp.sum(prod)
        res_v[...] = jnp.full((16,), score, _F32) + bfc_v[...]
        pltpu.sync_copy(res_v, out_hbm.at[wid])

    return seg_kernel(seg_idx, f1, w_fc_vec, b_fc16)


def kernel(text_feature, frame_features, segment_indices,
           W_gcn1, W_gcn2, gru_W_ih, gru_W_hh, gru_b_ih, gru_b_hh,
           h0_Wt, h0_Wf, h0_Wmf, h0_Wmt, h0_We2t, h0_We2f,
           h1_Wt, h1_Wf, h1_Wmf, h1_Wmt, h1_We2t, h1_We2f,
           W_fc, b_fc):
    f1 = pl.pallas_call(
        _mega_body,
        out_shape=jax.ShapeDtypeStruct((N, D_HID), _F32),
        scratch_shapes=[
            pltpu.VMEM((N, D_HID), _F32),
            pltpu.VMEM((N, D_HID), _F32),
            pltpu.VMEM((N, D_HID), _F32),
            pltpu.VMEM((N, 1), _F32),
            pltpu.VMEM((1, N), _F32),
            pltpu.VMEM((T_TOK, 3 * D_HID), _F32),
        ],
    )(frame_features, text_feature, W_gcn1, W_gcn2,
      gru_W_ih, gru_W_hh, gru_b_ih.reshape(1, -1), gru_b_hh.reshape(1, -1),
      h0_Wmf, h0_Wt, h0_We2t, h0_Wmt, h0_We2f, h0_Wf,
      h1_Wmt, h1_We2f, h1_Wf)

    seg_out = _seg_pool_sc(segment_indices.astype(jnp.int32), f1,
                           W_fc.reshape(-1), jnp.broadcast_to(b_fc, (16,)))
    return seg_out[:, 0]


# speedup vs baseline: 1.0399x; 1.0399x over previous
import functools
import math

import jax
import jax.numpy as jnp
from jax import lax
from jax.experimental import pallas as pl
from jax.experimental.pallas import tpu as pltpu
from jax.experimental.pallas import tpu_sc as plsc

N = 4096
D_FEAT = 128
D_HID = 64
T_TOK = 77
D_TXT = 768
N_SEG = 32
SEG_LEN = 64

ROW_BLK = 512
N_BLK = N // ROW_BLK

_F32 = jnp.float32


def _bf(x):
    return x.astype(jnp.bfloat16)


def _mm(a, b):
    return jax.lax.dot_general(a, b, (((1,), (0,)), ((), ())),
                               preferred_element_type=_F32)


def _mega_body(frames_ref, text_ref, w1_ref, w2_ref,
               wih_ref, whh_ref, bih_ref, bhh_ref,
               wmf_ref, wt0_ref, we2t0_ref, wmt0_ref, we2f0_ref, wf0_ref,
               wmt1_ref, we2f1_ref, wf1_ref,
               f1_out_ref,
               bi_s, y_s, wf_s, sqc_s, sqr_s, gi_s):
    xw = _mm(frames_ref[...], w1_ref[...])
    row = lax.broadcasted_iota(jnp.int32, (N, 1), 0)
    edge = (row == 0) | (row == N - 1)
    dinv = 1.0 / jnp.sqrt(jnp.where(edge, jnp.float32(2.0), jnp.float32(3.0)))
    zero = jnp.zeros((1, 1), _F32)
    dinv_up = jnp.concatenate([zero, dinv[:-1, :]], axis=0)
    dinv_dn = jnp.concatenate([dinv[1:, :], zero], axis=0)
    c_up = _bf(dinv * dinv_up).astype(_F32)
    c_self = _bf(dinv * dinv).astype(_F32)
    c_dn = _bf(dinv * dinv_dn).astype(_F32)
    xb = _bf(xw).astype(_F32)
    zrow = jnp.zeros((1, D_HID), _F32)
    x_up = jnp.concatenate([zrow, xb[:-1, :]], axis=0)
    x_dn = jnp.concatenate([xb[1:, :], zrow], axis=0)
    bi = jax.nn.relu(c_up * x_up + c_self * xb + c_dn * x_dn)
    bi_s[...] = bi
    y_s[...] = _mm(bi, w2_ref[...])
    sq = jnp.sum(bi * bi, axis=1)
    sqc_s[...] = sq[:, None]
    sqr_s[...] = sq[None, :]

    def dist_blk(i):
        off = pl.multiple_of(i * ROW_BLK, ROW_BLK)
        xb = bi_s[pl.ds(off, ROW_BLK), :]
        sqb = sqc_s[pl.ds(off, ROW_BLK), :]
        xxt = jax.lax.dot_general(xb, bi_s[...], (((1,), (1,)), ((), ())),
                                  preferred_element_type=_F32)
        d2 = sqb + sqr_s[...] - 2.0 * xxt
        return jnp.sqrt(jnp.maximum(d2, 0.0))

    def pass_a(i, dsum):
        return dsum + jnp.sum(dist_blk(i))

    dsum = lax.fori_loop(0, N_BLK, pass_a, jnp.float32(0.0), unroll=2)
    stat = dsum * (1.0 / (N * N))

    def pass_b(i, m0):
        off = pl.multiple_of(i * ROW_BLK, ROW_BLK)
        adj = jnp.exp(-dist_blk(i) / (stat + 1e-6))
        rowsum = jnp.sum(adj, axis=1, keepdims=True)
        adjn = adj / (rowsum + 1e-6)
        wfb = jax.nn.relu(_mm(adjn, y_s[...]))
        wf_s[pl.ds(off, ROW_BLK), :] = wfb
        return m0 + jnp.sum(jax.nn.relu(_mm(wfb, wmf_ref[...])),
                            axis=0, keepdims=True)

    m0 = lax.fori_loop(0, N_BLK, pass_b, jnp.zeros((1, D_HID), _F32),
                       unroll=2)

    gi_s[...] = _mm(text_ref[...], wih_ref[...]) + bih_ref[...]
    whh = whh_ref[...]
    bhh = bhh_ref[...]
    H = D_HID

    def gru_step(i, h):
        gi = gi_s[pl.ds(i, 1), :]
        gh = _mm(h, whh) + bhh
        r = jax.nn.sigmoid(gi[:, :H] + gh[:, :H])
        z = jax.nn.sigmoid(gi[:, H:2 * H] + gh[:, H:2 * H])
        n = jnp.tanh(gi[:, 2 * H:] + r * gh[:, 2 * H:])
        return (1.0 - z) * n + z * h

    t = lax.fori_loop(0, T_TOK, gru_step, jnp.zeros((1, H), _F32))

    m0 = m0 * (1.0 / N)
    t0 = jax.nn.relu(_mm(t, wt0_ref[...]) + _mm(m0, we2t0_ref[...]))
    bias0 = _mm(jax.nn.relu(_mm(t, wmt0_ref[...])), we2f0_ref[...])
    f0 = jax.nn.relu(_mm(wf_s[...], wf0_ref[...]) + bias0)
    bias1 = _mm(jax.nn.relu(_mm(t0, wmt1_ref[...])), we2f1_ref[...])
    f1_out_ref[...] = jax.nn.relu(_mm(f0, wf1_ref[...]) + bias1)


def _sc_round_bf16(v):
    u = lax.bitcast_convert_type(v, jnp.uint32)
    lsb = (u >> jnp.uint32(16)) & jnp.uint32(1)
    r = (u + jnp.uint32(0x7FFF) + lsb) & jnp.uint32(0xFFFF0000)
    return lax.bitcast_convert_type(r, _F32)


def _seg_pool_sc(seg_idx, f1, w_fc_vec, b_fc16):
    mesh = plsc.VectorSubcoreMesh(core_axis_name="c", subcore_axis_name="s")

    @functools.partial(
        pl.kernel, mesh=mesh,
        compiler_params=pltpu.CompilerParams(needs_layout_passes=False,
                                             use_tc_tiling_on_sc=False),
        out_type=jax.ShapeDtypeStruct((N_SEG, 16), _F32),
        scratch_types=[
            pltpu.VMEM((SEG_LEN,), jnp.int32),
            pltpu.VMEM((SEG_LEN, D_HID), _F32),
            pltpu.VMEM((D_HID,), _F32),
            pltpu.VMEM((16,), _F32),
            pltpu.VMEM((16,), _F32),
            pltpu.SemaphoreType.DMA,
        ],
    )
    def seg_kernel(idx_hbm, f_hbm, wfc_hbm, bfc_hbm, out_hbm,
                   idx_v, rows_v, wfc_v, bfc_v, res_v, sem):
        wid = lax.axis_index("s") * 2 + lax.axis_index("c")
        pltpu.sync_copy(wfc_hbm, wfc_v)
        pltpu.sync_copy(bfc_hbm, bfc_v)
        pltpu.sync_copy(idx_hbm.at[wid], idx_v)
        pltpu.async_copy(f_hbm.at[idx_v], rows_v, sem).wait()
        prod = jnp.zeros((16,), _F32)
        for c in range(D_HID // 16):
            cs = jnp.zeros((16,), _F32)
            for r in range(SEG_LEN):
                cs = cs + rows_v[r, pl.ds(c * 16, 16)]
            segb = _sc_round_bf16(cs * (1.0 / SEG_LEN))
            prod = prod + segb * _sc_round_bf16(wfc_v[pl.ds(c * 16, 16)])
        score = jnp.sum(prod)
        res_v[...] = jnp.full((16,), score, _F32) + bfc_v[...]
        pltpu.sync_copy(res_v, out_hbm.at[wid])

    return seg_kernel(seg_idx, f1, w_fc_vec, b_fc16)


def kernel(text_feature, frame_features, segment_indices,
           W_gcn1, W_gcn2, gru_W_ih, gru_W_hh, gru_b_ih, gru_b_hh,
           h0_Wt, h0_Wf, h0_Wmf, h0_Wmt, h0_We2t, h0_We2f,
           h1_Wt, h1_Wf, h1_Wmf, h1_Wmt, h1_We2t, h1_We2f,
           W_fc, b_fc):
    f1 = pl.pallas_call(
        _mega_body,
        out_shape=jax.ShapeDtypeStruct((N, D_HID), _F32),
        scratch_shapes=[
            pltpu.VMEM((N, D_HID), _F32),
            pltpu.VMEM((N, D_HID), _F32),
            pltpu.VMEM((N, D_HID), _F32),
            pltpu.VMEM((N, 1), _F32),
            pltpu.VMEM((1, N), _F32),
            pltpu.VMEM((T_TOK, 3 * D_HID), _F32),
        ],
    )(frame_features, text_feature, W_gcn1, W_gcn2,
      gru_W_ih, gru_W_hh, gru_b_ih.reshape(1, -1), gru_b_hh.reshape(1, -1),
      h0_Wmf, h0_Wt, h0_We2t, h0_Wmt, h0_We2f, h0_Wf,
      h1_Wmt, h1_We2f, h1_Wf)

    seg_out = _seg_pool_sc(segment_indices.astype(jnp.int32), f1,
                           W_fc.reshape(-1), jnp.broadcast_to(b_fc, (16,)))
    return seg_out[:, 0]


# speedup vs baseline: 1.1639x; 1.1193x over previous
import functools
import math

import jax
import jax.numpy as jnp
from jax import lax
from jax.experimental import pallas as pl
from jax.experimental.pallas import tpu as pltpu
from jax.experimental.pallas import tpu_sc as plsc

N = 4096
D_FEAT = 128
D_HID = 64
T_TOK = 77
D_TXT = 768
N_SEG = 32
SEG_LEN = 64

ROW_BLK = 512
N_BLK = N // ROW_BLK

_F32 = jnp.float32


def _bf(x):
    return x.astype(jnp.bfloat16)


def _mm(a, b):
    return jax.lax.dot_general(a, b, (((1,), (0,)), ((), ())),
                               preferred_element_type=_F32)


def _mega_body(frames_ref, text_ref, w1_ref, w2_ref,
               wih_ref, whh_ref, bih_ref, bhh_ref,
               wmf_ref, wt0_ref, we2t0_ref, wmt0_ref, we2f0_ref, wf0_ref,
               wmt1_ref, we2f1_ref, wf1_ref,
               f1_out_ref,
               bi_s, y_s, wf_s, sqc_s, sqr_s, gi_s):
    xw = _mm(frames_ref[...], w1_ref[...])
    row = lax.broadcasted_iota(jnp.int32, (N, 1), 0)
    edge = (row == 0) | (row == N - 1)
    dinv = 1.0 / jnp.sqrt(jnp.where(edge, jnp.float32(2.0), jnp.float32(3.0)))
    zero = jnp.zeros((1, 1), _F32)
    dinv_up = jnp.concatenate([zero, dinv[:-1, :]], axis=0)
    dinv_dn = jnp.concatenate([dinv[1:, :], zero], axis=0)
    c_up = _bf(dinv * dinv_up).astype(_F32)
    c_self = _bf(dinv * dinv).astype(_F32)
    c_dn = _bf(dinv * dinv_dn).astype(_F32)
    xb = _bf(xw).astype(_F32)
    zrow = jnp.zeros((1, D_HID), _F32)
    x_up = jnp.concatenate([zrow, xb[:-1, :]], axis=0)
    x_dn = jnp.concatenate([xb[1:, :], zrow], axis=0)
    bi = jax.nn.relu(c_up * x_up + c_self * xb + c_dn * x_dn)
    bi_s[...] = bi
    y_s[...] = _mm(bi, w2_ref[...])
    sq = jnp.sum(bi * bi, axis=1)
    sqc_s[...] = sq[:, None]
    sqr_s[...] = sq[None, :]

    def dist_blk(i):
        off = pl.multiple_of(i * ROW_BLK, ROW_BLK)
        xb = bi_s[pl.ds(off, ROW_BLK), :]
        sqb = sqc_s[pl.ds(off, ROW_BLK), :]
        xxt = jax.lax.dot_general(xb, bi_s[...], (((1,), (1,)), ((), ())),
                                  preferred_element_type=_F32)
        d2 = sqb + sqr_s[...] - 2.0 * xxt
        return jnp.sqrt(jnp.maximum(d2, 0.0))

    dsum = jnp.float32(0.0)
    for i in range(N_BLK):
        xi = bi_s[pl.ds(i * ROW_BLK, ROW_BLK), :]
        sqb_i = sqc_s[pl.ds(i * ROW_BLK, ROW_BLK), :]
        for j in range(i, N_BLK):
            xj = bi_s[pl.ds(j * ROW_BLK, ROW_BLK), :]
            sqr_j = sqr_s[:, pl.ds(j * ROW_BLK, ROW_BLK)]
            xxt = jax.lax.dot_general(xi, xj, (((1,), (1,)), ((), ())),
                                      preferred_element_type=_F32)
            d2 = sqb_i + sqr_j - 2.0 * xxt
            s = jnp.sum(jnp.sqrt(jnp.maximum(d2, 0.0)))
            dsum = dsum + (s if i == j else 2.0 * s)
    stat = dsum * (1.0 / (N * N))

    def pass_b(i, m0):
        off = pl.multiple_of(i * ROW_BLK, ROW_BLK)
        adj = jnp.exp(-dist_blk(i) / (stat + 1e-6))
        rowsum = jnp.sum(adj, axis=1, keepdims=True)
        adjn = adj / (rowsum + 1e-6)
        wfb = jax.nn.relu(_mm(adjn, y_s[...]))
        wf_s[pl.ds(off, ROW_BLK), :] = wfb
        return m0 + jnp.sum(jax.nn.relu(_mm(wfb, wmf_ref[...])),
                            axis=0, keepdims=True)

    m0 = lax.fori_loop(0, N_BLK, pass_b, jnp.zeros((1, D_HID), _F32),
                       unroll=2)

    gi_s[...] = _mm(text_ref[...], wih_ref[...]) + bih_ref[...]
    whh = whh_ref[...]
    bhh = bhh_ref[...]
    H = D_HID

    def gru_step(i, h):
        gi = gi_s[pl.ds(i, 1), :]
        gh = _mm(h, whh) + bhh
        r = jax.nn.sigmoid(gi[:, :H] + gh[:, :H])
        z = jax.nn.sigmoid(gi[:, H:2 * H] + gh[:, H:2 * H])
        n = jnp.tanh(gi[:, 2 * H:] + r * gh[:, 2 * H:])
        return (1.0 - z) * n + z * h

    t = lax.fori_loop(0, T_TOK, gru_step, jnp.zeros((1, H), _F32))

    m0 = m0 * (1.0 / N)
    t0 = jax.nn.relu(_mm(t, wt0_ref[...]) + _mm(m0, we2t0_ref[...]))
    bias0 = _mm(jax.nn.relu(_mm(t, wmt0_ref[...])), we2f0_ref[...])
    f0 = jax.nn.relu(_mm(wf_s[...], wf0_ref[...]) + bias0)
    bias1 = _mm(jax.nn.relu(_mm(t0, wmt1_ref[...])), we2f1_ref[...])
    f1_out_ref[...] = jax.nn.relu(_mm(f0, wf1_ref[...]) + bias1)


def _sc_round_bf16(v):
    u = lax.bitcast_convert_type(v, jnp.uint32)
    lsb = (u >> jnp.uint32(16)) & jnp.uint32(1)
    r = (u + jnp.uint32(0x7FFF) + lsb) & jnp.uint32(0xFFFF0000)
    return lax.bitcast_convert_type(r, _F32)


def _seg_pool_sc(seg_idx, f1, w_fc_vec, b_fc16):
    mesh = plsc.VectorSubcoreMesh(core_axis_name="c", subcore_axis_name="s")

    @functools.partial(
        pl.kernel, mesh=mesh,
        compiler_params=pltpu.CompilerParams(needs_layout_passes=False,
                                             use_tc_tiling_on_sc=False),
        out_type=jax.ShapeDtypeStruct((N_SEG, 16), _F32),
        scratch_types=[
            pltpu.VMEM((SEG_LEN,), jnp.int32),
            pltpu.VMEM((SEG_LEN, D_HID), _F32),
            pltpu.VMEM((D_HID,), _F32),
            pltpu.VMEM((16,), _F32),
            pltpu.VMEM((16,), _F32),
            pltpu.SemaphoreType.DMA,
        ],
    )
    def seg_kernel(idx_hbm, f_hbm, wfc_hbm, bfc_hbm, out_hbm,
                   idx_v, rows_v, wfc_v, bfc_v, res_v, sem):
        wid = lax.axis_index("s") * 2 + lax.axis_index("c")
        pltpu.sync_copy(wfc_hbm, wfc_v)
        pltpu.sync_copy(bfc_hbm, bfc_v)
        pltpu.sync_copy(idx_hbm.at[wid], idx_v)
        pltpu.async_copy(f_hbm.at[idx_v], rows_v, sem).wait()
        prod = jnp.zeros((16,), _F32)
        for c in range(D_HID // 16):
            cs = jnp.zeros((16,), _F32)
            for r in range(SEG_LEN):
                cs = cs + rows_v[r, pl.ds(c * 16, 16)]
            segb = _sc_round_bf16(cs * (1.0 / SEG_LEN))
            prod = prod + segb * _sc_round_bf16(wfc_v[pl.ds(c * 16, 16)])
        score = jnp.sum(prod)
        res_v[...] = jnp.full((16,), score, _F32) + bfc_v[...]
        pltpu.sync_copy(res_v, out_hbm.at[wid])

    return seg_kernel(seg_idx, f1, w_fc_vec, b_fc16)


def kernel(text_feature, frame_features, segment_indices,
           W_gcn1, W_gcn2, gru_W_ih, gru_W_hh, gru_b_ih, gru_b_hh,
           h0_Wt, h0_Wf, h0_Wmf, h0_Wmt, h0_We2t, h0_We2f,
           h1_Wt, h1_Wf, h1_Wmf, h1_Wmt, h1_We2t, h1_We2f,
           W_fc, b_fc):
    f1 = pl.pallas_call(
        _mega_body,
        out_shape=jax.ShapeDtypeStruct((N, D_HID), _F32),
        scratch_shapes=[
            pltpu.VMEM((N, D_HID), _F32),
            pltpu.VMEM((N, D_HID), _F32),
            pltpu.VMEM((N, D_HID), _F32),
            pltpu.VMEM((N, 1), _F32),
            pltpu.VMEM((1, N), _F32),
            pltpu.VMEM((T_TOK, 3 * D_HID), _F32),
        ],
    )(frame_features, text_feature, W_gcn1, W_gcn2,
      gru_W_ih, gru_W_hh, gru_b_ih.reshape(1, -1), gru_b_hh.reshape(1, -1),
      h0_Wmf, h0_Wt, h0_We2t, h0_Wmt, h0_We2f, h0_Wf,
      h1_Wmt, h1_We2f, h1_Wf)

    seg_out = _seg_pool_sc(segment_indices.astype(jnp.int32), f1,
                           W_fc.reshape(-1), jnp.broadcast_to(b_fc, (16,)))
    return seg_out[:, 0]
